# baseline (speedup 1.0000x reference)
import functools
import math

import jax
import jax.numpy as jnp
from jax import lax
from jax.experimental import pallas as pl
from jax.experimental.pallas import tpu as pltpu
from jax.experimental.pallas import tpu_sc as plsc

B = 16384
NUM_CORES = 1
NUM_SUBCORES = 16
NUM_WORKERS = NUM_CORES * NUM_SUBCORES
B_PER_W = B // NUM_WORKERS
LANES = 16
SCALE = math.log(10.0) / 400.0

_mesh = plsc.VectorSubcoreMesh(core_axis_name="c", subcore_axis_name="s",
                               num_cores=NUM_CORES)


@functools.partial(
    pl.kernel,
    mesh=_mesh,
    out_type=jax.ShapeDtypeStruct((B,), jnp.float32),
    scratch_types=[
        pltpu.VMEM((B_PER_W,), jnp.int32),
        pltpu.VMEM((B_PER_W,), jnp.int32),
        pltpu.VMEM((B_PER_W,), jnp.float32),
        pltpu.VMEM((B_PER_W,), jnp.float32),
        pltpu.SemaphoreType.DMA,
        pltpu.SemaphoreType.DMA,
        pltpu.SemaphoreType.DMA,
        pltpu.SemaphoreType.DMA,
        pltpu.SemaphoreType.DMA,
        pltpu.SemaphoreType.DMA,
        pltpu.SemaphoreType.DMA,
    ],
)
def _elo_sc(rating_hbm, home_hbm, away_hbm, out_hbm,
            hidx, aidx, hval, aval, hisem, aisem, hsem, asem, hsem1, asem1, osem):
    wid = lax.axis_index("s") * NUM_CORES + lax.axis_index("c")
    base = wid * B_PER_W
    half = B_PER_W // 2
    hicp = pltpu.async_copy(home_hbm.at[pl.ds(base, B_PER_W)], hidx, hisem)
    aicp = pltpu.async_copy(away_hbm.at[pl.ds(base, B_PER_W)], aidx, aisem)
    hicp.wait()
    hcp0 = pltpu.async_copy(rating_hbm.at[hidx.at[pl.ds(0, half)]],
                            hval.at[pl.ds(0, half)], hsem)
    aicp.wait()
    acp0 = pltpu.async_copy(rating_hbm.at[aidx.at[pl.ds(0, half)]],
                            aval.at[pl.ds(0, half)], asem)
    hcp1 = pltpu.async_copy(rating_hbm.at[hidx.at[pl.ds(half, half)]],
                            hval.at[pl.ds(half, half)], hsem1)
    acp1 = pltpu.async_copy(rating_hbm.at[aidx.at[pl.ds(half, half)]],
                            aval.at[pl.ds(half, half)], asem1)
    hcp0.wait()
    acp0.wait()
    for i in range(half // LANES):
        sl = pl.ds(i * LANES, LANES)
        x = (hval[sl] - aval[sl]) * SCALE
        hval[sl] = 1.0 / (1.0 + jnp.exp(x))
    ocp0 = pltpu.async_copy(hval.at[pl.ds(0, half)],
                            out_hbm.at[pl.ds(base, half)], osem)
    hcp1.wait()
    acp1.wait()
    for i in range(half // LANES, B_PER_W // LANES):
        sl = pl.ds(i * LANES, LANES)
        x = (hval[sl] - aval[sl]) * SCALE
        hval[sl] = 1.0 / (1.0 + jnp.exp(x))
    ocp1 = pltpu.async_copy(hval.at[pl.ds(half, half)],
                            out_hbm.at[pl.ds(base + half, half)], osem)
    ocp0.wait()
    ocp1.wait()


def kernel(rating, home, away):
    return _elo_sc(rating, home.astype(jnp.int32), away.astype(jnp.int32))


# speedup vs baseline: 1.0262x; 1.0262x over previous
import functools
import math

import jax
import jax.numpy as jnp
from jax import lax
from jax.experimental import pallas as pl
from jax.experimental.pallas import tpu as pltpu
from jax.experimental.pallas import tpu_sc as plsc

B = 16384
NUM_CORES = 2
NUM_SUBCORES = 16
NUM_WORKERS = NUM_CORES * NUM_SUBCORES
B_PER_W = B // NUM_WORKERS
LANES = 16
SCALE = math.log(10.0) / 400.0

_mesh = plsc.VectorSubcoreMesh(core_axis_name="c", subcore_axis_name="s")


@functools.partial(
    pl.kernel,
    mesh=_mesh,
    out_type=jax.ShapeDtypeStruct((B,), jnp.float32),
    scratch_types=[
        pltpu.VMEM((B_PER_W,), jnp.int32),
        pltpu.VMEM((B_PER_W,), jnp.int32),
        pltpu.VMEM((B_PER_W,), jnp.float32),
        pltpu.VMEM((B_PER_W,), jnp.float32),
        pltpu.SemaphoreType.DMA,
        pltpu.SemaphoreType.DMA,
        pltpu.SemaphoreType.DMA,
        pltpu.SemaphoreType.DMA,
        pltpu.SemaphoreType.DMA,
        pltpu.SemaphoreType.DMA,
        pltpu.SemaphoreType.DMA,
    ],
)
def _elo_sc(rating_hbm, home_hbm, away_hbm, out_hbm,
            hidx, aidx, hval, aval, hisem, aisem, hsem, asem, hsem1, asem1, osem):
    wid = lax.axis_index("s") * NUM_CORES + lax.axis_index("c")
    base = wid * B_PER_W
    half = B_PER_W // 2
    with jax.named_scope("idx_fire"):
        hicp = pltpu.async_copy(home_hbm.at[pl.ds(base, B_PER_W)], hidx, hisem)
        aicp = pltpu.async_copy(away_hbm.at[pl.ds(base, B_PER_W)], aidx, aisem)
    with jax.named_scope("idx_wait"):
        hicp.wait()
        aicp.wait()
    with jax.named_scope("gather_fire"):
        hcp0 = pltpu.async_copy(rating_hbm.at[hidx.at[pl.ds(0, half)]],
                                hval.at[pl.ds(0, half)], hsem)
        acp0 = pltpu.async_copy(rating_hbm.at[aidx.at[pl.ds(0, half)]],
                                aval.at[pl.ds(0, half)], asem)
        hcp1 = pltpu.async_copy(rating_hbm.at[hidx.at[pl.ds(half, half)]],
                                hval.at[pl.ds(half, half)], hsem1)
        acp1 = pltpu.async_copy(rating_hbm.at[aidx.at[pl.ds(half, half)]],
                                aval.at[pl.ds(half, half)], asem1)
    with jax.named_scope("gather0_wait"):
        hcp0.wait()
        acp0.wait()
    with jax.named_scope("compute0"):
        for i in range(half // LANES):
            sl = pl.ds(i * LANES, LANES)
            x = (hval[sl] - aval[sl]) * SCALE
            hval[sl] = 1.0 / (1.0 + jnp.exp(x))
        ocp0 = pltpu.async_copy(hval.at[pl.ds(0, half)],
                                out_hbm.at[pl.ds(base, half)], osem)
    with jax.named_scope("gather1_wait"):
        hcp1.wait()
        acp1.wait()
    with jax.named_scope("compute1"):
        for i in range(half // LANES, B_PER_W // LANES):
            sl = pl.ds(i * LANES, LANES)
            x = (hval[sl] - aval[sl]) * SCALE
            hval[sl] = 1.0 / (1.0 + jnp.exp(x))
        ocp1 = pltpu.async_copy(hval.at[pl.ds(half, half)],
                                out_hbm.at[pl.ds(base + half, half)], osem)
    with jax.named_scope("out_wait"):
        ocp0.wait()
        ocp1.wait()


def kernel(rating, home, away):
    return _elo_sc(rating, home.astype(jnp.int32), away.astype(jnp.int32))


# speedup vs baseline: 1.0352x; 1.0088x over previous
import functools
import math

import jax
import jax.numpy as jnp
from jax import lax
from jax.experimental import pallas as pl
from jax.experimental.pallas import tpu as pltpu
from jax.experimental.pallas import tpu_sc as plsc

B = 16384
NUM_CORES = 2
NUM_SUBCORES = 16
NUM_WORKERS = NUM_CORES * NUM_SUBCORES
B_PER_W = B // NUM_WORKERS
HALF = B_PER_W // 2
LANES = 16
SCALE = math.log(10.0) / 400.0

_mesh = plsc.VectorSubcoreMesh(core_axis_name="c", subcore_axis_name="s")


@functools.partial(
    pl.kernel,
    mesh=_mesh,
    out_type=jax.ShapeDtypeStruct((B,), jnp.float32),
    scratch_types=[
        pltpu.VMEM((B_PER_W,), jnp.int32),
        pltpu.VMEM((B_PER_W,), jnp.int32),
        pltpu.VMEM((B_PER_W,), jnp.float32),
        pltpu.VMEM((B_PER_W,), jnp.float32),
        pltpu.SemaphoreType.DMA,
        pltpu.SemaphoreType.DMA,
        pltpu.SemaphoreType.DMA,
        pltpu.SemaphoreType.DMA,
        pltpu.SemaphoreType.DMA,
        pltpu.SemaphoreType.DMA,
        pltpu.SemaphoreType.DMA,
    ],
)
def _elo_sc(rating_hbm, home_hbm, away_hbm, out_hbm,
            hidx, aidx, hval, aval, hisem, aisem, hsem, asem, hsem1, asem1,
            osem):
    wid = lax.axis_index("s") * NUM_CORES + lax.axis_index("c")
    base = wid * B_PER_W
    hicp = pltpu.async_copy(home_hbm.at[pl.ds(base, B_PER_W)], hidx, hisem)
    aicp = pltpu.async_copy(away_hbm.at[pl.ds(base, B_PER_W)], aidx, aisem)
    hicp.wait()
    hcp0 = pltpu.async_copy(rating_hbm.at[hidx.at[pl.ds(0, HALF)]],
                            hval.at[pl.ds(0, HALF)], hsem)
    aicp.wait()
    acp0 = pltpu.async_copy(rating_hbm.at[aidx.at[pl.ds(0, HALF)]],
                            aval.at[pl.ds(0, HALF)], asem)
    hcp1 = pltpu.async_copy(rating_hbm.at[hidx.at[pl.ds(HALF, HALF)]],
                            hval.at[pl.ds(HALF, HALF)], hsem1)
    acp1 = pltpu.async_copy(rating_hbm.at[aidx.at[pl.ds(HALF, HALF)]],
                            aval.at[pl.ds(HALF, HALF)], asem1)

    def sigmoid_chunk(i, _):
        sl = pl.ds(i * LANES, LANES)
        x = (hval[sl] - aval[sl]) * SCALE
        hval[sl] = 1.0 / (1.0 + jnp.exp(x))
        return 0

    hcp0.wait()
    acp0.wait()
    lax.fori_loop(0, HALF // LANES, sigmoid_chunk, 0)
    ocp0 = pltpu.async_copy(hval.at[pl.ds(0, HALF)],
                            out_hbm.at[pl.ds(base, HALF)], osem)
    hcp1.wait()
    acp1.wait()
    lax.fori_loop(HALF // LANES, B_PER_W // LANES, sigmoid_chunk, 0)
    ocp1 = pltpu.async_copy(hval.at[pl.ds(HALF, HALF)],
                            out_hbm.at[pl.ds(base + HALF, HALF)], osem)
    ocp0.wait()
    ocp1.wait()


def kernel(rating, home, away):
    return _elo_sc(rating, home.astype(jnp.int32), away.astype(jnp.int32))
